# Initial kernel scaffold; baseline (speedup 1.0000x reference)
import jax
import jax.numpy as jnp
from jax.experimental import pallas as pl


def kernel(z, codebook):
    raise NotImplementedError("write your pallas kernel here")


# speedup vs baseline: 1.3424x; 1.3424x over previous
import functools

import jax
import jax.numpy as jnp
from jax import lax
from jax.experimental import pallas as pl
from jax.experimental.pallas import tpu as pltpu
from jax.experimental.pallas import tpu_sc as plsc

K = 8192
D = 32
BETA = 0.5
N = 64 * 1024

BLK = 256
KBLK = 2048
NKB = K // KBLK
NBLK = N // BLK


def _vq_tc_body(zt_ref, cb_ref, cn_ref, tok_ref, acc_ref):
    i = pl.program_id(0)
    zt = zt_ref[...]
    zn = jnp.sum(zt * zt, axis=0, keepdims=True)
    zt_r = zt.astype(jnp.bfloat16).astype(jnp.float32)

    best = jnp.full((1, BLK), jnp.inf, jnp.float32)
    bidx = jnp.zeros((1, BLK), jnp.int32)
    for j in range(NKB):
        cb = cb_ref[pl.ds(j * KBLK, KBLK), :]
        cn = cn_ref[pl.ds(j * KBLK, KBLK), :]
        zc = lax.dot_general(cb, zt_r, (((1,), (0,)), ((), ())),
                             preferred_element_type=jnp.float32)
        dist = (zn - 2.0 * zc) + cn
        m = jnp.min(dist, axis=0, keepdims=True)
        iota = lax.broadcasted_iota(jnp.int32, (KBLK, BLK), 0)
        idx = jnp.min(jnp.where(dist == m, iota, K), axis=0,
                      keepdims=True) + j * KBLK
        take = m < best
        best = jnp.where(take, m, best)
        bidx = jnp.where(take, idx, bidx)

    tok_ref[...] = bidx.reshape(1, 1, BLK)

    @pl.when(i == 0)
    def _():
        acc_ref[...] = jnp.zeros((1, 1), jnp.float32)
    acc_ref[...] += jnp.sum(best).reshape(1, 1)


def _vq_argmin(z_t, codebook, cn):
    return pl.pallas_call(
        _vq_tc_body,
        grid=(NBLK,),
        in_specs=[
            pl.BlockSpec((D, BLK), lambda i: (0, i)),
            pl.BlockSpec((K, D), lambda i: (0, 0)),
            pl.BlockSpec((K, 1), lambda i: (0, 0)),
        ],
        out_specs=[
            pl.BlockSpec((1, 1, BLK), lambda i: (i, 0, 0)),
            pl.BlockSpec((1, 1), lambda i: (0, 0)),
        ],
        out_shape=[
            jax.ShapeDtypeStruct((NBLK, 1, BLK), jnp.int32),
            jax.ShapeDtypeStruct((1, 1), jnp.float32),
        ],
    )(z_t, codebook, cn)


def _sc_gather(codebook, idx3, nc, ns):
    nw = nc * ns
    b_per_w = N // nw
    ch = 128
    nch = b_per_w // ch
    mesh = plsc.VectorSubcoreMesh(core_axis_name="c", subcore_axis_name="s")

    @functools.partial(
        pl.kernel, mesh=mesh,
        compiler_params=pltpu.CompilerParams(use_tc_tiling_on_sc=False),
        out_type=jax.ShapeDtypeStruct((N, D), jnp.float32),
        scratch_types=[
            pltpu.VMEM((nch, ch), jnp.int32),
            pltpu.VMEM((b_per_w, D), jnp.float32),
            pltpu.SemaphoreType.DMA,
        ],
    )
    def gather_k(cb_hbm, idx_hbm, out_hbm, idx_v, rows_v, sem):
        wid = lax.axis_index("s") * nc + lax.axis_index("c")
        pltpu.sync_copy(idx_hbm.at[wid], idx_v)
        copies = []
        for j in range(nch):
            copies.append(pltpu.async_copy(
                cb_hbm.at[idx_v.at[j]],
                rows_v.at[pl.ds(j * ch, ch)], sem))
        for c in copies:
            c.wait()
        pltpu.sync_copy(rows_v, out_hbm.at[pl.ds(wid * b_per_w, b_per_w)])

    return gather_k(codebook, idx3)


def kernel(z, codebook):
    z_t = z.reshape(-1, D).T
    cn = jnp.sum(codebook ** 2, axis=1)[:, None]
    tok3, acc = _vq_argmin(z_t, codebook, cn)
    tok_flat = tok3.reshape(-1)

    info = plsc.get_sparse_core_info()
    nc, ns = info.num_cores, info.num_subcores
    idx3 = tok_flat.reshape(nc * ns, (N // (nc * ns)) // 128, 128)
    z_q = _sc_gather(codebook, idx3, nc, ns).reshape(z.shape)

    commit_loss = (BETA / float(N * D)) * acc[0, 0]
    tokens = tok_flat.reshape(z.shape[:-1])
    return (z_q, tokens, commit_loss)
